# baseline (speedup 1.0000x reference)
import functools

import jax
import jax.numpy as jnp
from jax import lax
from jax.experimental import pallas as pl
from jax.experimental.pallas import tpu as pltpu
from jax.experimental.pallas import tpu_sc as plsc

_N = 32
_F = 3 * 512 * 512
_TOTAL = _N * _F
_COPY = (_N - 1) * _F
_NW = 32
_CHUNK = _COPY // _NW
_XCHUNK = _F // _NW
_K = 16
_B = _CHUNK // _K

_mesh = plsc.VectorSubcoreMesh(core_axis_name="c", subcore_axis_name="s")


@functools.partial(
    pl.kernel,
    mesh=_mesh,
    out_type=jax.ShapeDtypeStruct((_TOTAL,), jnp.float32),
    scratch_types=[
        pltpu.SemaphoreType.DMA,
        pltpu.SemaphoreType.DMA,
    ],
)
def _ring_update(x_hbm, t_hbm, out_hbm, sem, sx):
    wid = lax.axis_index("s") * 2 + lax.axis_index("c")
    base = pl.multiple_of(wid * _CHUNK, 8)
    xb = pl.multiple_of(wid * _XCHUNK, 8)

    def copy_k(k):
        return pltpu.make_async_copy(
            t_hbm.at[pl.ds(_F + base + k * _B, _B)],
            out_hbm.at[pl.ds(base + k * _B, _B)], sem)

    x_copy = pltpu.make_async_copy(
        x_hbm.at[pl.ds(xb, _XCHUNK)],
        out_hbm.at[pl.ds(_COPY + xb, _XCHUNK)], sx)
    x_copy.start()
    for k in range(_K):
        copy_k(k).start()
    for k in range(_K):
        copy_k(k).wait()
    x_copy.wait()


def kernel(x, tensors):
    out = _ring_update(x.reshape(-1), tensors.reshape(-1))
    return out.reshape(tensors.shape)


# speedup vs baseline: 49.9846x; 49.9846x over previous
import jax
import jax.numpy as jnp
from jax.experimental import pallas as pl

_N = 32
_R = 3 * 512
_W = 512


def _shift_body(x_ref, t_ref, o_ref):
    i = pl.program_id(0)

    @pl.when(i < _N - 1)
    def _():
        o_ref[...] = t_ref[...]

    @pl.when(i == _N - 1)
    def _():
        o_ref[...] = x_ref[...]


def kernel(x, tensors):
    x2 = x.reshape(_R, _W)
    t2 = tensors.reshape(_N * _R, _W)
    out = pl.pallas_call(
        _shift_body,
        grid=(_N,),
        in_specs=[
            pl.BlockSpec((_R, _W), lambda i: (0, 0)),
            pl.BlockSpec((_R, _W), lambda i: (jnp.minimum(i + 1, _N - 1), 0)),
        ],
        out_specs=pl.BlockSpec((_R, _W), lambda i: (i, 0)),
        out_shape=jax.ShapeDtypeStruct((_N * _R, _W), jnp.float32),
    )(x2, t2)
    return out.reshape(tensors.shape)
